# baseline (speedup 1.0000x reference)
import functools
import jax
import jax.numpy as jnp
from jax import lax
from jax.experimental import pallas as pl
from jax.experimental.pallas import tpu as pltpu
from jax.experimental.pallas import tpu_sc as plsc

B_, L_, DIM_ = 4, 4096, 1024
NC, NS, LANES = 2, 16, 16
NW = NC * NS
WPB = NW // B_
L_PER_W = L_ // WPB
R_ = 16
NCH = L_PER_W // R_


def _sc_body(x_hbm, emb_hbm, out_hbm, emb_v, xb_v, si0, si1, so0, so1):
    wid = lax.axis_index("s") * NC + lax.axis_index("c")
    bidx = wid // WPB
    row0 = (wid % WPB) * L_PER_W
    sin = (si0, si1)
    sout = (so0, so1)

    def in_copies(slot, c):
        base = row0 + c * R_
        yield pltpu.make_async_copy(emb_hbm.at[pl.ds(base, R_)],
                                    emb_v.at[slot], sin[slot])
        yield pltpu.make_async_copy(x_hbm.at[bidx, pl.ds(base, R_)],
                                    xb_v.at[slot], sin[slot])

    def out_copies(slot, c):
        base = row0 + c * R_
        yield pltpu.make_async_copy(xb_v.at[slot],
                                    out_hbm.at[bidx, pl.ds(base, R_)],
                                    sout[slot])

    def compute(slot):
        eslot = emb_v.at[slot]
        xslot = xb_v.at[slot]

        def rbody(r, rcarry):
            for j in range(DIM_ // LANES):
                sl = pl.ds(j * LANES, LANES)
                plsc.addupdate(xslot.at[r, sl], eslot[r, sl])
            return rcarry

        lax.fori_loop(0, R_, rbody, 0)

    def step(slot, c, first, last):
        if not first:
            for d in out_copies(1 - slot, c - 1):
                d.wait()
        if not last:
            for d in in_copies(1 - slot, c + 1):
                d.start()
        for d in in_copies(slot, c):
            d.wait()
        compute(slot)
        for d in out_copies(slot, c):
            d.start()

    for d in in_copies(0, 0):
        d.start()
    step(0, 0, first=True, last=False)

    @pl.loop(0, (NCH - 2) // 2)
    def _steady(it):
        for k in range(2):
            c = 1 + it * 2 + k
            step((1 + k) % 2, c, first=False, last=False)

    step(1, NCH - 1, first=False, last=True)
    for d in out_copies(1, NCH - 1):
        d.wait()


def kernel(x, emb):
    mesh = plsc.VectorSubcoreMesh(core_axis_name="c", subcore_axis_name="s")
    return pl.kernel(
        _sc_body,
        out_type=jax.ShapeDtypeStruct((B_, L_, DIM_), jnp.float32),
        mesh=mesh,
        scratch_types=[
            pltpu.VMEM((2, R_, DIM_), jnp.float32),
            pltpu.VMEM((2, R_, DIM_), jnp.float32),
            pltpu.SemaphoreType.DMA,
            pltpu.SemaphoreType.DMA,
            pltpu.SemaphoreType.DMA,
            pltpu.SemaphoreType.DMA,
        ],
    )(x, emb)
